# Initial kernel scaffold; baseline (speedup 1.0000x reference)
import jax
import jax.numpy as jnp
from jax.experimental import pallas as pl


def kernel(x, es):
    raise NotImplementedError("write your pallas kernel here")


# speedup vs baseline: 5.7548x; 5.7548x over previous
import functools

import jax
import jax.numpy as jnp
from jax import lax
from jax.experimental import pallas as pl
from jax.experimental.pallas import tpu as pltpu
from jax.experimental.pallas import tpu_sc as plsc

N = 10000
E = 320000
D = 128
DP = 144
NW = 32
CHUNK = 128
CHUNKS_PW = 80
EPW = CHUNK * CHUNKS_PW
E_PAD = EPW * NW
ACC_ROWS = 10240
ROWS_PT = ACC_ROWS // 16
TRASH = N

_mesh = plsc.VectorSubcoreMesh(core_axis_name="c", subcore_axis_name="s")


@functools.partial(
    pl.kernel,
    mesh=_mesh,
    compiler_params=pltpu.CompilerParams(use_tc_tiling_on_sc=False),
    out_type=jax.ShapeDtypeStruct((2, ACC_ROWS, DP), jnp.float32),
    scratch_types=[
        pltpu.VMEM((CHUNKS_PW, CHUNK), jnp.int32),
        pltpu.VMEM((CHUNKS_PW, CHUNK), jnp.int32),
        pltpu.VMEM((CHUNK, DP), jnp.float32),
        pltpu.VMEM_SHARED((ACC_ROWS, DP), jnp.float32),
        pltpu.SemaphoreType.DMA,
    ],
)
def _sc_accumulate(xext, rowi, coli, out, ridx, cidx, rows, acc, sem):
    c = lax.axis_index("c")
    s = lax.axis_index("s")
    wid = c * 16 + s

    pltpu.sync_copy(rowi.at[wid], ridx)
    pltpu.sync_copy(coli.at[wid], cidx)

    zv = jnp.zeros((16,), jnp.float32)

    def zrow(i, carry):
        for j in range(DP // 16):
            rows[i, pl.ds(j * 16, 16)] = zv
        return carry

    lax.fori_loop(0, CHUNK, zrow, 0)
    for j in range(ROWS_PT // CHUNK):
        pltpu.sync_copy(rows, acc.at[pl.ds(s * ROWS_PT + j * CHUNK, CHUNK)])
    plsc.subcore_barrier()

    def body(j, carry):
        pltpu.async_copy(xext.at[ridx.at[j]], rows, sem).wait()
        pltpu.sync_copy(rows, acc.at[cidx.at[j]], add=True)
        return carry

    lax.fori_loop(0, CHUNKS_PW, body, 0)
    plsc.subcore_barrier()

    for j in range(ROWS_PT // CHUNK):
        off = s * ROWS_PT + j * CHUNK
        pltpu.sync_copy(acc.at[pl.ds(off, CHUNK)], rows)
        pltpu.sync_copy(rows, out.at[c, pl.ds(off, CHUNK)])


BN = 400


def _finish_body(p_ref, x_ref, o_ref):
    p0 = p_ref[0]
    p1 = p_ref[1]
    cnt = p0[:, D:D + 1] + p1[:, D:D + 1]
    sums = p0[:, :D] + p1[:, :D]
    o_ref[:, :D] = sums / jnp.maximum(cnt, 1.0)
    o_ref[:, D:] = jnp.where(cnt > 0.0, x_ref[...], 0.0)


_finish = pl.pallas_call(
    _finish_body,
    grid=(N // BN,),
    in_specs=[
        pl.BlockSpec((2, BN, DP), lambda i: (0, i, 0)),
        pl.BlockSpec((BN, D), lambda i: (i, 0)),
    ],
    out_specs=pl.BlockSpec((BN, 2 * D), lambda i: (i, 0)),
    out_shape=jax.ShapeDtypeStruct((N, 2 * D), jnp.float32),
)


@jax.jit
def kernel(x, es):
    col = es[0]
    row = es[1]
    xext = jnp.concatenate(
        [x,
         jnp.ones((N, 1), jnp.float32),
         jnp.zeros((N, DP - D - 1), jnp.float32)], axis=1)
    pad = E_PAD - E
    rowp = jnp.concatenate([row, jnp.zeros((pad,), jnp.int32)])
    colp = jnp.concatenate([col, jnp.full((pad,), TRASH, jnp.int32)])
    partial = _sc_accumulate(
        xext,
        rowp.reshape(NW, CHUNKS_PW, CHUNK),
        colp.reshape(NW, CHUNKS_PW, CHUNK),
    )
    return _finish(partial, x)


# speedup vs baseline: 7.7579x; 1.3481x over previous
import functools

import jax
import jax.numpy as jnp
from jax import lax
from jax.experimental import pallas as pl
from jax.experimental.pallas import tpu as pltpu
from jax.experimental.pallas import tpu_sc as plsc

N = 10000
E = 320000
D = 128
DP = 144
NW = 32
CHUNK = 64
CHUNKS_PW = 159
EPW = CHUNK * CHUNKS_PW
E_PAD = EPW * NW
ACC_ROWS = 10240
ROWS_PT = ACC_ROWS // 16
TRASH = N
K = 3
P = 2

_mesh = plsc.VectorSubcoreMesh(core_axis_name="c", subcore_axis_name="s")


@functools.partial(
    pl.kernel,
    mesh=_mesh,
    compiler_params=pltpu.CompilerParams(use_tc_tiling_on_sc=False),
    out_type=jax.ShapeDtypeStruct((2, ACC_ROWS, DP), jnp.float32),
    scratch_types=[
        pltpu.VMEM((CHUNKS_PW, CHUNK), jnp.int32),
        [pltpu.VMEM((2, CHUNK), jnp.int32)] * K,
        [pltpu.VMEM((CHUNK, DP), jnp.float32)] * K,
        pltpu.VMEM_SHARED((ACC_ROWS, DP), jnp.float32),
        [pltpu.SemaphoreType.DMA] * K,
        [pltpu.SemaphoreType.DMA] * K,
    ],
)
def _sc_accumulate(xext, exi, out, packed, idxu, rows, acc, gsem, ssem):
    c = lax.axis_index("c")
    s = lax.axis_index("s")
    wid = c * 16 + s

    pltpu.sync_copy(exi.at[wid], packed)

    zv = jnp.zeros((16,), jnp.float32)

    def zrow(i, carry):
        for j in range(DP // 16):
            rows[0][i, pl.ds(j * 16, 16)] = zv
        return carry

    lax.fori_loop(0, CHUNK, zrow, 0)
    for j in range(ROWS_PT // CHUNK):
        pltpu.sync_copy(rows[0], acc.at[pl.ds(s * ROWS_PT + j * CHUNK, CHUNK)])
    plsc.subcore_barrier()

    def unpack(t, b):
        for k in range(CHUNK // 16):
            v = packed[t, pl.ds(k * 16, 16)]
            idxu[b][0, pl.ds(k * 16, 16)] = v & 0xFFFF
            idxu[b][1, pl.ds(k * 16, 16)] = v >> 16

    def gather(t, b):
        return pltpu.make_async_copy(xext.at[idxu[b].at[0]], rows[b], gsem[b])

    def scatter(t, b):
        return pltpu.make_async_copy(rows[b], acc.at[idxu[b].at[1]], ssem[b])

    for t in range(P):
        unpack(t, t % K)
        gather(t, t % K).start()

    def body(jj, carry):
        for u in range(K):
            b = u % K
            j = jj * K + u
            gather(j, b).wait()
            scatter(j, b).start(add=True)
            b2 = (u + P) % K

            @pl.when(j + P < CHUNKS_PW)
            def _():
                @pl.when(j + P >= K)
                def _():
                    scatter(j + P - K, b2).wait()
                unpack(j + P, b2)
                gather(j + P, b2).start()

        return carry

    lax.fori_loop(0, CHUNKS_PW // K, body, 0)
    for u in range(K):
        j = CHUNKS_PW - K + u
        scatter(j, j % K).wait()
    plsc.subcore_barrier()

    for j in range(ROWS_PT // CHUNK):
        off = s * ROWS_PT + j * CHUNK
        pltpu.sync_copy(acc.at[pl.ds(off, CHUNK)], rows[0])
        pltpu.sync_copy(rows[0], out.at[c, pl.ds(off, CHUNK)])


BN = 400


def _finish_body(p_ref, x_ref, o_ref):
    p0 = p_ref[0]
    p1 = p_ref[1]
    cnt = p0[:, D:D + 1] + p1[:, D:D + 1]
    sums = p0[:, :D] + p1[:, :D]
    o_ref[:, :D] = sums / jnp.maximum(cnt, 1.0)
    o_ref[:, D:] = jnp.where(cnt > 0.0, x_ref[...], 0.0)


_finish = pl.pallas_call(
    _finish_body,
    grid=(N // BN,),
    in_specs=[
        pl.BlockSpec((2, BN, DP), lambda i: (0, i, 0)),
        pl.BlockSpec((BN, D), lambda i: (i, 0)),
    ],
    out_specs=pl.BlockSpec((BN, 2 * D), lambda i: (i, 0)),
    out_shape=jax.ShapeDtypeStruct((N, 2 * D), jnp.float32),
)


@jax.jit
def kernel(x, es):
    col = es[0]
    row = es[1]
    xext = jnp.concatenate(
        [x,
         jnp.ones((N, 1), jnp.float32),
         jnp.zeros((N, DP - D - 1), jnp.float32)], axis=1)
    pad = E_PAD - E
    rowp = jnp.concatenate([row, jnp.zeros((pad,), jnp.int32)])
    colp = jnp.concatenate([col, jnp.full((pad,), TRASH, jnp.int32)])
    packed = jnp.bitwise_or(jnp.left_shift(colp, 16), rowp)
    partial = _sc_accumulate(xext, packed.reshape(NW, CHUNKS_PW, CHUNK))
    return _finish(partial, x)
